# Initial kernel scaffold; baseline (speedup 1.0000x reference)
import jax
import jax.numpy as jnp
from jax.experimental import pallas as pl


def kernel(x, edge_index, W1, b1, W2, b2):
    raise NotImplementedError("write your pallas kernel here")


# speedup vs baseline: 34.7347x; 34.7347x over previous
import functools

import jax
import jax.numpy as jnp
from jax import lax
from jax.experimental import pallas as pl
from jax.experimental.pallas import tpu as pltpu
from jax.experimental.pallas import tpu_sc as plsc

N_NODES = 100000
N_EDGES = 3200000
NPAD = 102400
NROW, NCOL = 200, 512
NW = 32
BLK = 128
NBPW = 782
EPAD = NW * NBPW * BLK
RPS = NPAD // 16

_mesh = plsc.VectorSubcoreMesh(core_axis_name="c", subcore_axis_name="s")


def _edge_pass(src, dst, tables):
    P = max(len(tables), 1)
    with_gather = bool(tables)

    scratch = [
        pltpu.VMEM((BLK,), jnp.int32),
        pltpu.VMEM((BLK,), jnp.int32),
        pltpu.VMEM((RPS,), jnp.float32),
    ]
    scratch += [pltpu.VMEM((BLK,), jnp.float32) for _ in range(P)]
    scratch += [pltpu.VMEM_SHARED((NPAD,), jnp.float32) for _ in range(P)]
    if with_gather:
        scratch += [pltpu.VMEM_SHARED((NPAD,), jnp.float32) for _ in range(P)]

    @functools.partial(
        pl.kernel,
        out_type=[jax.ShapeDtypeStruct((2 * NPAD,), jnp.float32)] * P,
        mesh=_mesh,
        scratch_types=scratch,
    )
    def k(*refs):
        src_hbm, dst_hbm = refs[0], refs[1]
        tab_hbm = refs[2:2 + len(tables)]
        z_hbm = refs[2 + len(tables)]
        pos = 3 + len(tables)
        out_hbm = refs[pos:pos + P]; pos += P
        sidx, didx, bounce = refs[pos:pos + 3]; pos += 3
        vals = refs[pos:pos + P]; pos += P
        acc = refs[pos:pos + P]; pos += P
        tab_sh = refs[pos:pos + P] if with_gather else ()

        c = lax.axis_index("c")
        s = lax.axis_index("s")
        wid = s * 2 + c
        sl = pl.ds(s * RPS, RPS)

        pltpu.sync_copy(z_hbm.at[sl], bounce)
        for p in range(P):
            pltpu.sync_copy(bounce, acc[p].at[sl])
        for p in range(len(tables)):
            pltpu.sync_copy(tab_hbm[p].at[sl], bounce)
            pltpu.sync_copy(bounce, tab_sh[p].at[sl])
        if not with_gather:
            @pl.loop(0, BLK, step=16)
            def _(i):
                vals[0][pl.ds(i, 16)] = jnp.ones((16,), jnp.float32)
        plsc.subcore_barrier()

        @pl.loop(0, NBPW)
        def _(i):
            off = (wid * NBPW + i) * BLK
            pltpu.sync_copy(dst_hbm.at[pl.ds(off, BLK)], didx)
            if with_gather:
                pltpu.sync_copy(src_hbm.at[pl.ds(off, BLK)], sidx)
                for p in range(P):
                    pltpu.sync_copy(tab_sh[p].at[sidx], vals[p])
            for p in range(P):
                pltpu.sync_copy(vals[p], acc[p].at[didx], add=True)

        plsc.subcore_barrier()
        for p in range(P):
            pltpu.sync_copy(acc[p].at[sl], bounce)
            pltpu.sync_copy(bounce, out_hbm[p].at[pl.ds(c * NPAD + s * RPS, RPS)])

    zeros = jnp.zeros((NPAD,), jnp.float32)
    outs = k(src, dst, *tables, zeros)
    return outs if isinstance(outs, (list, tuple)) else [outs]


_TGRID = NROW // 8
_pln = lambda: pl.BlockSpec((8, NCOL), lambda i: (i, 0))
_pln2 = lambda: pl.BlockSpec((2, 8, NCOL), lambda i: (0, i, 0))
_pln3 = lambda: pl.BlockSpec((3, 8, NCOL), lambda i: (0, i, 0))


def _t1(degp, x3):
    def body(degp_ref, x3_ref, dinv_ref, u3_ref):
        deg = degp_ref[0] + degp_ref[1] + 1.0
        dinv = lax.rsqrt(deg)
        dinv_ref[...] = dinv
        for kk in range(3):
            u3_ref[kk] = x3_ref[kk] * dinv

    return pl.pallas_call(
        body,
        grid=(_TGRID,),
        in_specs=[_pln2(), _pln3()],
        out_specs=[_pln(), _pln3()],
        out_shape=[jax.ShapeDtypeStruct((NROW, NCOL), jnp.float32),
                   jax.ShapeDtypeStruct((3, NROW, NCOL), jnp.float32)],
    )(degp, x3)


def _t2(vp3, dinv, x3, W1, b1r, W2):
    def body(vp_ref, dinv_ref, x3_ref, w1_ref, b1_ref, w2_ref,
             tabc_ref, z_ref):
        dinv = dinv_ref[...]
        w1 = w1_ref[...]
        b1 = b1_ref[...]
        w2 = w2_ref[...]
        y = [dinv * (vp_ref[0, kk] + vp_ref[1, kk]) + dinv * dinv * x3_ref[kk]
             for kk in range(3)]
        z = jnp.zeros_like(dinv)
        for j in range(16):
            hj = y[0] * w1[0, j] + y[1] * w1[1, j] + y[2] * w1[2, j] + b1[0, j]
            rh = jnp.maximum(hj, 0.0).astype(jnp.bfloat16).astype(jnp.float32)
            z = z + rh * w2[j, 0]
        z_ref[...] = z
        tabc_ref[...] = dinv * z

    wspec = lambda shp: pl.BlockSpec(shp, lambda i: tuple(0 for _ in shp))
    return pl.pallas_call(
        body,
        grid=(_TGRID,),
        in_specs=[pl.BlockSpec((2, 3, 8, NCOL), lambda i: (0, 0, i, 0)),
                  _pln(), _pln3(),
                  wspec((3, 16)), wspec((1, 16)), wspec((16, 1))],
        out_specs=[_pln(), _pln()],
        out_shape=[jax.ShapeDtypeStruct((NROW, NCOL), jnp.float32)] * 2,
    )(vp3, dinv, x3, W1, b1r, W2)


def _t3(wp, dinv, z, b2r):
    def body(wp_ref, dinv_ref, z_ref, b2_ref, out_ref):
        dinv = dinv_ref[...]
        out_ref[...] = (dinv * (wp_ref[0] + wp_ref[1])
                        + dinv * dinv * z_ref[...] + b2_ref[0, 0])

    return pl.pallas_call(
        body,
        grid=(_TGRID,),
        in_specs=[_pln2(), _pln(), _pln(),
                  pl.BlockSpec((1, 1), lambda i: (0, 0))],
        out_specs=_pln(),
        out_shape=jax.ShapeDtypeStruct((NROW, NCOL), jnp.float32),
    )(wp, dinv, z, b2r)


def kernel(x, edge_index, W1, b1, W2, b2):
    ei = edge_index.astype(jnp.int32)
    epad = jnp.full((EPAD - N_EDGES,), N_NODES, jnp.int32)
    src = jnp.concatenate([ei[0], epad])
    dst = jnp.concatenate([ei[1], epad])
    _r = lambda a: a.astype(jnp.bfloat16).astype(jnp.float32)
    x = _r(x)
    W1 = _r(W1)
    W2 = _r(W2)
    x3 = jnp.pad(x.T, ((0, 0), (0, NPAD - N_NODES)))

    degp = _edge_pass(dst, dst, [])[0].reshape(2, NROW, NCOL)
    dinv, u3 = _t1(degp, x3.reshape(3, NROW, NCOL))
    u_planes = [u3.reshape(3, NPAD)[kk] for kk in range(3)]
    vp = _edge_pass(src, dst, u_planes)
    vp3 = jnp.stack([v.reshape(2, NROW, NCOL) for v in vp], axis=1)
    tabc, z = _t2(vp3, dinv, x3.reshape(3, NROW, NCOL),
                  W1, b1.reshape(1, 16), W2)
    wp = _edge_pass(src, dst, [tabc.reshape(NPAD)])[0].reshape(2, NROW, NCOL)
    out = _t3(wp, dinv, z, b2.reshape(1, 1))
    return out.reshape(NPAD)[:N_NODES].reshape(N_NODES, 1)


# speedup vs baseline: 120.5940x; 3.4719x over previous
import functools

import jax
import jax.numpy as jnp
from jax import lax
from jax.experimental import pallas as pl
from jax.experimental.pallas import tpu as pltpu
from jax.experimental.pallas import tpu_sc as plsc

N_NODES = 100000
N_EDGES = 3200000
NPAD = 102400
NROW, NCOL = 200, 512
NW = 32
BLK = 128
CH = 8
NCHUNK = 98
NBPW = CH * NCHUNK
EPAD = NW * NBPW * BLK
EROWS = EPAD // BLK
RPS = NPAD // 16

_mesh = plsc.VectorSubcoreMesh(core_axis_name="c", subcore_axis_name="s")


def _edge_pass(src, dst, tables):
    P = max(len(tables), 1)
    with_gather = bool(tables)

    scratch = [
        pltpu.VMEM((CH, BLK), jnp.int32),
        pltpu.VMEM((CH, BLK), jnp.int32),
        pltpu.VMEM((RPS,), jnp.float32),
    ]
    scratch += [pltpu.VMEM((CH, BLK), jnp.float32) for _ in range(P)]
    scratch += [pltpu.VMEM_SHARED((NPAD,), jnp.float32) for _ in range(P)]
    if with_gather:
        scratch += [pltpu.VMEM_SHARED((NPAD,), jnp.float32) for _ in range(P)]
    scratch += [pltpu.SemaphoreType.DMA, pltpu.SemaphoreType.DMA]

    @functools.partial(
        pl.kernel,
        out_type=[jax.ShapeDtypeStruct((2 * NPAD,), jnp.float32)] * P,
        mesh=_mesh,
        scratch_types=scratch,
    )
    def k(*refs):
        src_hbm, dst_hbm = refs[0], refs[1]
        tab_hbm = refs[2:2 + len(tables)]
        z_hbm = refs[2 + len(tables)]
        pos = 3 + len(tables)
        out_hbm = refs[pos:pos + P]; pos += P
        sidx, didx, bounce = refs[pos:pos + 3]; pos += 3
        vals = refs[pos:pos + P]; pos += P
        acc = refs[pos:pos + P]; pos += P
        if with_gather:
            tab_sh = refs[pos:pos + P]; pos += P
        else:
            tab_sh = ()
        sem_g, sem_s = refs[pos:pos + 2]

        c = lax.axis_index("c")
        s = lax.axis_index("s")
        wid = s * 2 + c
        sl = pl.ds(s * RPS, RPS)

        pltpu.sync_copy(z_hbm.at[sl], bounce)
        for p in range(P):
            pltpu.sync_copy(bounce, acc[p].at[sl])
        for p in range(len(tables)):
            pltpu.sync_copy(tab_hbm[p].at[sl], bounce)
            pltpu.sync_copy(bounce, tab_sh[p].at[sl])
        if not with_gather:
            @pl.loop(0, CH)
            def _(j):
                @pl.loop(0, BLK, step=16)
                def _(i):
                    vals[0][j, pl.ds(i, 16)] = jnp.ones((16,), jnp.float32)
        plsc.subcore_barrier()

        @pl.loop(0, NCHUNK)
        def _(t):
            row = (wid * NCHUNK + t) * CH
            pltpu.sync_copy(dst_hbm.at[pl.ds(row, CH)], didx)
            if with_gather:
                pltpu.sync_copy(src_hbm.at[pl.ds(row, CH)], sidx)
                hs = [pltpu.async_copy(tab_sh[p].at[sidx.at[j]],
                                       vals[p].at[j], sem_g)
                      for j in range(CH) for p in range(P)]
                for h in hs:
                    h.wait()
            hs = [pltpu.async_copy(vals[p].at[j], acc[p].at[didx.at[j]],
                                   sem_s, add=True)
                  for j in range(CH) for p in range(P)]
            for h in hs:
                h.wait()

        plsc.subcore_barrier()
        for p in range(P):
            pltpu.sync_copy(acc[p].at[sl], bounce)
            pltpu.sync_copy(bounce, out_hbm[p].at[pl.ds(c * NPAD + s * RPS, RPS)])

    zeros = jnp.zeros((NPAD,), jnp.float32)
    outs = k(src, dst, *tables, zeros)
    return outs if isinstance(outs, (list, tuple)) else [outs]


_TGRID = NROW // 8
_pln = lambda: pl.BlockSpec((8, NCOL), lambda i: (i, 0))
_pln2 = lambda: pl.BlockSpec((2, 8, NCOL), lambda i: (0, i, 0))
_pln3 = lambda: pl.BlockSpec((3, 8, NCOL), lambda i: (0, i, 0))


def _t1(degp, x3):
    def body(degp_ref, x3_ref, dinv_ref, u3_ref):
        deg = degp_ref[0] + degp_ref[1] + 1.0
        dinv = lax.rsqrt(deg)
        dinv_ref[...] = dinv
        for kk in range(3):
            u3_ref[kk] = x3_ref[kk] * dinv

    return pl.pallas_call(
        body,
        grid=(_TGRID,),
        in_specs=[_pln2(), _pln3()],
        out_specs=[_pln(), _pln3()],
        out_shape=[jax.ShapeDtypeStruct((NROW, NCOL), jnp.float32),
                   jax.ShapeDtypeStruct((3, NROW, NCOL), jnp.float32)],
    )(degp, x3)


def _t2(vp3, dinv, x3, W1, b1r, W2):
    def body(vp_ref, dinv_ref, x3_ref, w1_ref, b1_ref, w2_ref,
             tabc_ref, z_ref):
        dinv = dinv_ref[...]
        w1 = w1_ref[...]
        b1 = b1_ref[...]
        w2 = w2_ref[...]
        y = [dinv * (vp_ref[0, kk] + vp_ref[1, kk]) + dinv * dinv * x3_ref[kk]
             for kk in range(3)]
        z = jnp.zeros_like(dinv)
        for j in range(16):
            hj = y[0] * w1[0, j] + y[1] * w1[1, j] + y[2] * w1[2, j] + b1[0, j]
            rh = jnp.maximum(hj, 0.0).astype(jnp.bfloat16).astype(jnp.float32)
            z = z + rh * w2[j, 0]
        z_ref[...] = z
        tabc_ref[...] = dinv * z

    wspec = lambda shp: pl.BlockSpec(shp, lambda i: tuple(0 for _ in shp))
    return pl.pallas_call(
        body,
        grid=(_TGRID,),
        in_specs=[pl.BlockSpec((2, 3, 8, NCOL), lambda i: (0, 0, i, 0)),
                  _pln(), _pln3(),
                  wspec((3, 16)), wspec((1, 16)), wspec((16, 1))],
        out_specs=[_pln(), _pln()],
        out_shape=[jax.ShapeDtypeStruct((NROW, NCOL), jnp.float32)] * 2,
    )(vp3, dinv, x3, W1, b1r, W2)


def _t3(wp, dinv, z, b2r):
    def body(wp_ref, dinv_ref, z_ref, b2_ref, out_ref):
        dinv = dinv_ref[...]
        out_ref[...] = (dinv * (wp_ref[0] + wp_ref[1])
                        + dinv * dinv * z_ref[...] + b2_ref[0, 0])

    return pl.pallas_call(
        body,
        grid=(_TGRID,),
        in_specs=[_pln2(), _pln(), _pln(),
                  pl.BlockSpec((1, 1), lambda i: (0, 0))],
        out_specs=_pln(),
        out_shape=jax.ShapeDtypeStruct((NROW, NCOL), jnp.float32),
    )(wp, dinv, z, b2r)


def kernel(x, edge_index, W1, b1, W2, b2):
    ei = edge_index.astype(jnp.int32)
    epad = jnp.full((EPAD - N_EDGES,), N_NODES, jnp.int32)
    src = jnp.concatenate([ei[0], epad]).reshape(EROWS, BLK)
    dst = jnp.concatenate([ei[1], epad]).reshape(EROWS, BLK)
    _r = lambda a: a.astype(jnp.bfloat16).astype(jnp.float32)
    x = _r(x)
    W1 = _r(W1)
    W2 = _r(W2)
    x3 = jnp.pad(x.T, ((0, 0), (0, NPAD - N_NODES)))

    degp = _edge_pass(dst, dst, [])[0].reshape(2, NROW, NCOL)
    dinv, u3 = _t1(degp, x3.reshape(3, NROW, NCOL))
    u_planes = [u3.reshape(3, NPAD)[kk] for kk in range(3)]
    vp = _edge_pass(src, dst, u_planes)
    vp3 = jnp.stack([v.reshape(2, NROW, NCOL) for v in vp], axis=1)
    tabc, z = _t2(vp3, dinv, x3.reshape(3, NROW, NCOL),
                  W1, b1.reshape(1, 16), W2)
    wp = _edge_pass(src, dst, [tabc.reshape(NPAD)])[0].reshape(2, NROW, NCOL)
    out = _t3(wp, dinv, z, b2.reshape(1, 1))
    return out.reshape(NPAD)[:N_NODES].reshape(N_NODES, 1)


# speedup vs baseline: 120.8434x; 1.0021x over previous
import functools

import jax
import jax.numpy as jnp
from jax import lax
from jax.experimental import pallas as pl
from jax.experimental.pallas import tpu as pltpu
from jax.experimental.pallas import tpu_sc as plsc

N_NODES = 100000
N_EDGES = 3200000
NPAD = 102400
NROW, NCOL = 200, 512
NW = 32
BLK = 128
CH = 8
NCHUNK = 98
NBPW = CH * NCHUNK
EPAD = NW * NBPW * BLK
EROWS = EPAD // BLK
RPS = NPAD // 16

_mesh = plsc.VectorSubcoreMesh(core_axis_name="c", subcore_axis_name="s")


def _edge_pass(src, dst, tables):
    P = max(len(tables), 1)
    with_gather = bool(tables)

    scratch = [
        pltpu.VMEM((CH, BLK), jnp.int32),
        pltpu.VMEM((CH, BLK), jnp.int32),
        pltpu.VMEM((RPS,), jnp.float32),
    ]
    scratch += [pltpu.VMEM((CH, BLK), jnp.float32) for _ in range(P)]
    scratch += [pltpu.VMEM_SHARED((NPAD,), jnp.float32) for _ in range(P)]
    if with_gather:
        scratch += [pltpu.VMEM_SHARED((NPAD,), jnp.float32) for _ in range(P)]
    scratch += [pltpu.SemaphoreType.DMA, pltpu.SemaphoreType.DMA]

    @functools.partial(
        pl.kernel,
        out_type=[jax.ShapeDtypeStruct((2 * NPAD,), jnp.float32)] * P,
        mesh=_mesh,
        scratch_types=scratch,
    )
    def k(*refs):
        src_hbm, dst_hbm = refs[0], refs[1]
        tab_hbm = refs[2:2 + len(tables)]
        z_hbm = refs[2 + len(tables)]
        pos = 3 + len(tables)
        out_hbm = refs[pos:pos + P]; pos += P
        sidx, didx, bounce = refs[pos:pos + 3]; pos += 3
        vals = refs[pos:pos + P]; pos += P
        acc = refs[pos:pos + P]; pos += P
        if with_gather:
            tab_sh = refs[pos:pos + P]; pos += P
        else:
            tab_sh = ()
        sem_g, sem_s = refs[pos:pos + 2]

        c = lax.axis_index("c")
        s = lax.axis_index("s")
        wid = s * 2 + c
        sl = pl.ds(s * RPS, RPS)

        pltpu.sync_copy(z_hbm.at[sl], bounce)
        for p in range(P):
            pltpu.sync_copy(bounce, acc[p].at[sl])
        for p in range(len(tables)):
            pltpu.sync_copy(tab_hbm[p].at[sl], bounce)
            pltpu.sync_copy(bounce, tab_sh[p].at[sl])
        if not with_gather:
            @pl.loop(0, CH)
            def _(j):
                @pl.loop(0, BLK, step=16)
                def _(i):
                    vals[0][j, pl.ds(i, 16)] = jnp.ones((16,), jnp.float32)
        plsc.subcore_barrier()

        @pl.loop(0, NCHUNK)
        def _(t):
            row = (wid * NCHUNK + t) * CH
            pltpu.sync_copy(dst_hbm.at[pl.ds(row, CH)], didx)
            if with_gather:
                pltpu.sync_copy(src_hbm.at[pl.ds(row, CH)], sidx)
                hs = [pltpu.async_copy(tab_sh[p].at[sidx.at[j]],
                                       vals[p].at[j], sem_g)
                      for j in range(CH) for p in range(P)]
                for h in hs:
                    h.wait()
            hs = [pltpu.async_copy(vals[p].at[j], acc[p].at[didx.at[j]],
                                   sem_s, add=True)
                  for j in range(CH) for p in range(P)]
            for h in hs:
                h.wait()

        plsc.subcore_barrier()
        for p in range(P):
            pltpu.sync_copy(acc[p].at[sl], bounce)
            pltpu.sync_copy(bounce, out_hbm[p].at[pl.ds(c * NPAD + s * RPS, RPS)])

    zeros = jnp.zeros((NPAD,), jnp.float32)
    outs = k(src, dst, *tables, zeros)
    return outs if isinstance(outs, (list, tuple)) else [outs]


_TGRID = NROW // 8
_pln = lambda: pl.BlockSpec((8, NCOL), lambda i: (i, 0))
_pln2 = lambda: pl.BlockSpec((2, 8, NCOL), lambda i: (0, i, 0))
_pln3 = lambda: pl.BlockSpec((3, 8, NCOL), lambda i: (0, i, 0))


def _t1(degp, x3):
    def body(degp_ref, x3_ref, dinv_ref, u3_ref):
        deg = degp_ref[0] + degp_ref[1] + 1.0
        r = lax.rsqrt(deg)
        dinv = r * (1.5 - 0.5 * deg * r * r)
        dinv_ref[...] = dinv
        for kk in range(3):
            u3_ref[kk] = x3_ref[kk] * dinv

    return pl.pallas_call(
        body,
        grid=(_TGRID,),
        in_specs=[_pln2(), _pln3()],
        out_specs=[_pln(), _pln3()],
        out_shape=[jax.ShapeDtypeStruct((NROW, NCOL), jnp.float32),
                   jax.ShapeDtypeStruct((3, NROW, NCOL), jnp.float32)],
    )(degp, x3)


def _t2(vp3, dinv, x3, W1, b1r, W2):
    def body(vp_ref, dinv_ref, x3_ref, w1_ref, b1_ref, w2_ref,
             tabc_ref, z_ref):
        dinv = dinv_ref[...]
        w1 = w1_ref[...]
        b1 = b1_ref[...]
        w2 = w2_ref[...]
        y = [dinv * (vp_ref[0, kk] + vp_ref[1, kk]) + dinv * dinv * x3_ref[kk]
             for kk in range(3)]
        z = jnp.zeros_like(dinv)
        for j in range(16):
            hj = y[0] * w1[0, j] + y[1] * w1[1, j] + y[2] * w1[2, j] + b1[0, j]
            z = z + jnp.maximum(hj, 0.0) * w2[j, 0]
        z_ref[...] = z
        tabc_ref[...] = dinv * z

    wspec = lambda shp: pl.BlockSpec(shp, lambda i: tuple(0 for _ in shp))
    return pl.pallas_call(
        body,
        grid=(_TGRID,),
        in_specs=[pl.BlockSpec((2, 3, 8, NCOL), lambda i: (0, 0, i, 0)),
                  _pln(), _pln3(),
                  wspec((3, 16)), wspec((1, 16)), wspec((16, 1))],
        out_specs=[_pln(), _pln()],
        out_shape=[jax.ShapeDtypeStruct((NROW, NCOL), jnp.float32)] * 2,
    )(vp3, dinv, x3, W1, b1r, W2)


def _t3(wp, dinv, z, b2r):
    def body(wp_ref, dinv_ref, z_ref, b2_ref, out_ref):
        dinv = dinv_ref[...]
        out_ref[...] = (dinv * (wp_ref[0] + wp_ref[1])
                        + dinv * dinv * z_ref[...] + b2_ref[0, 0])

    return pl.pallas_call(
        body,
        grid=(_TGRID,),
        in_specs=[_pln2(), _pln(), _pln(),
                  pl.BlockSpec((1, 1), lambda i: (0, 0))],
        out_specs=_pln(),
        out_shape=jax.ShapeDtypeStruct((NROW, NCOL), jnp.float32),
    )(wp, dinv, z, b2r)


def kernel(x, edge_index, W1, b1, W2, b2):
    ei = edge_index.astype(jnp.int32)
    epad = jnp.full((EPAD - N_EDGES,), N_NODES, jnp.int32)
    src = jnp.concatenate([ei[0], epad]).reshape(EROWS, BLK)
    dst = jnp.concatenate([ei[1], epad]).reshape(EROWS, BLK)
    def _r(a):
        t = jax.lax.bitcast_convert_type(a, jnp.uint32)
        t = ((t + jnp.uint32(0x7FFF) + ((t >> 16) & jnp.uint32(1)))
             & jnp.uint32(0xFFFF0000))
        return jax.lax.bitcast_convert_type(t, jnp.float32)
    W1 = _r(W1)
    W2 = _r(W2)
    x3 = jnp.pad(x.T, ((0, 0), (0, NPAD - N_NODES)))

    degp = _edge_pass(dst, dst, [])[0].reshape(2, NROW, NCOL)
    dinv, u3 = _t1(degp, x3.reshape(3, NROW, NCOL))
    u_planes = [u3.reshape(3, NPAD)[kk] for kk in range(3)]
    vp = _edge_pass(src, dst, u_planes)
    vp3 = jnp.stack([v.reshape(2, NROW, NCOL) for v in vp], axis=1)
    tabc, z = _t2(vp3, dinv, x3.reshape(3, NROW, NCOL),
                  W1, b1.reshape(1, 16), W2)
    wp = _edge_pass(src, dst, [tabc.reshape(NPAD)])[0].reshape(2, NROW, NCOL)
    out = _t3(wp, dinv, z, b2.reshape(1, 1))
    return out.reshape(NPAD)[:N_NODES].reshape(N_NODES, 1)


# speedup vs baseline: 157.7996x; 1.3058x over previous
import functools

import jax
import jax.numpy as jnp
from jax import lax
from jax.experimental import pallas as pl
from jax.experimental.pallas import tpu as pltpu
from jax.experimental.pallas import tpu_sc as plsc

N_NODES = 100000
N_EDGES = 3200000
NPAD = 102400
NROW, NCOL = 200, 512
NW = 32
BLK = 128
CH = 8
NCHUNK = 98
NBPW = CH * NCHUNK
EPAD = NW * NBPW * BLK
EROWS = EPAD // BLK
RPS = NPAD // 16

_mesh = plsc.VectorSubcoreMesh(core_axis_name="c", subcore_axis_name="s")


def _edge_pass(src, dst, tables):
    P = max(len(tables), 1)
    with_gather = bool(tables)

    scratch = [
        pltpu.VMEM((2 * CH, BLK), jnp.int32),
        pltpu.VMEM((2 * CH, BLK), jnp.int32),
        pltpu.VMEM((RPS,), jnp.float32),
    ]
    scratch += [pltpu.VMEM((2 * CH, BLK), jnp.float32) for _ in range(P)]
    scratch += [pltpu.VMEM_SHARED((NPAD,), jnp.float32) for _ in range(P)]
    if with_gather:
        scratch += [pltpu.VMEM_SHARED((NPAD,), jnp.float32) for _ in range(P)]
    scratch += [pltpu.SemaphoreType.DMA, pltpu.SemaphoreType.DMA,
                pltpu.SemaphoreType.DMA]

    @functools.partial(
        pl.kernel,
        out_type=[jax.ShapeDtypeStruct((2 * NPAD,), jnp.float32)] * P,
        mesh=_mesh,
        scratch_types=scratch,
    )
    def k(*refs):
        src_hbm, dst_hbm = refs[0], refs[1]
        tab_hbm = refs[2:2 + len(tables)]
        z_hbm = refs[2 + len(tables)]
        pos = 3 + len(tables)
        out_hbm = refs[pos:pos + P]; pos += P
        sidx, didx, bounce = refs[pos:pos + 3]; pos += 3
        vals = refs[pos:pos + P]; pos += P
        acc = refs[pos:pos + P]; pos += P
        if with_gather:
            tab_sh = refs[pos:pos + P]; pos += P
        else:
            tab_sh = ()
        sem_i, sem_g, sem_s = refs[pos:pos + 3]

        c = lax.axis_index("c")
        s = lax.axis_index("s")
        wid = s * 2 + c
        sl = pl.ds(s * RPS, RPS)

        pltpu.sync_copy(z_hbm.at[sl], bounce)
        for p in range(P):
            pltpu.sync_copy(bounce, acc[p].at[sl])
        for p in range(len(tables)):
            pltpu.sync_copy(tab_hbm[p].at[sl], bounce)
            pltpu.sync_copy(bounce, tab_sh[p].at[sl])
        if not with_gather:
            @pl.loop(0, 2 * CH)
            def _(j):
                @pl.loop(0, BLK, step=16)
                def _(i):
                    vals[0][j, pl.ds(i, 16)] = jnp.ones((16,), jnp.float32)
        plsc.subcore_barrier()

        def _load_idx(row, half):
            hs = [pltpu.async_copy(dst_hbm.at[pl.ds(row, CH)],
                                   didx.at[pl.ds(half * CH, CH)], sem_i)]
            if with_gather:
                hs.append(pltpu.async_copy(src_hbm.at[pl.ds(row, CH)],
                                           sidx.at[pl.ds(half * CH, CH)],
                                           sem_i))
            return hs

        def _gathers(half):
            return [pltpu.async_copy(tab_sh[p].at[sidx.at[j]],
                                     vals[p].at[j], sem_g)
                    for j in range(half * CH, half * CH + CH)
                    for p in range(P)]

        def _scatters(half):
            return [pltpu.async_copy(vals[p].at[j], acc[p].at[didx.at[j]],
                                     sem_s, add=True)
                    for j in range(half * CH, half * CH + CH)
                    for p in range(P)]

        @pl.loop(0, NCHUNK // 2)
        def _(t):
            row = (wid * NCHUNK + 2 * t) * CH
            hia = _load_idx(row, 0)
            hib = _load_idx(row + CH, 1)
            for h in hia:
                h.wait()
            if with_gather:
                ga = _gathers(0)
            for h in hib:
                h.wait()
            if with_gather:
                for h in ga:
                    h.wait()
            sa = _scatters(0)
            if with_gather:
                gb = _gathers(1)
                for h in gb:
                    h.wait()
            sb = _scatters(1)
            for h in sa + sb:
                h.wait()

        plsc.subcore_barrier()
        for p in range(P):
            pltpu.sync_copy(acc[p].at[sl], bounce)
            pltpu.sync_copy(bounce, out_hbm[p].at[pl.ds(c * NPAD + s * RPS, RPS)])

    zeros = jnp.zeros((NPAD,), jnp.float32)
    outs = k(src, dst, *tables, zeros)
    return outs if isinstance(outs, (list, tuple)) else [outs]


_TGRID = NROW // 8
_pln = lambda: pl.BlockSpec((8, NCOL), lambda i: (i, 0))
_pln2 = lambda: pl.BlockSpec((2, 8, NCOL), lambda i: (0, i, 0))
_pln3 = lambda: pl.BlockSpec((3, 8, NCOL), lambda i: (0, i, 0))


def _t1(degp, x3):
    def body(degp_ref, x3_ref, dinv_ref, u3_ref):
        deg = degp_ref[0] + degp_ref[1] + 1.0
        r = lax.rsqrt(deg)
        dinv = r * (1.5 - 0.5 * deg * r * r)
        dinv_ref[...] = dinv
        for kk in range(3):
            u3_ref[kk] = x3_ref[kk] * dinv

    return pl.pallas_call(
        body,
        grid=(_TGRID,),
        in_specs=[_pln2(), _pln3()],
        out_specs=[_pln(), _pln3()],
        out_shape=[jax.ShapeDtypeStruct((NROW, NCOL), jnp.float32),
                   jax.ShapeDtypeStruct((3, NROW, NCOL), jnp.float32)],
    )(degp, x3)


def _t2(vp3, dinv, x3, W1, b1r, W2):
    def body(vp_ref, dinv_ref, x3_ref, w1_ref, b1_ref, w2_ref,
             tabc_ref, z_ref):
        dinv = dinv_ref[...]
        w1 = w1_ref[...]
        b1 = b1_ref[...]
        w2 = w2_ref[...]
        y = [dinv * (vp_ref[0, kk] + vp_ref[1, kk]) + dinv * dinv * x3_ref[kk]
             for kk in range(3)]
        z = jnp.zeros_like(dinv)
        for j in range(16):
            hj = y[0] * w1[0, j] + y[1] * w1[1, j] + y[2] * w1[2, j] + b1[0, j]
            z = z + jnp.maximum(hj, 0.0) * w2[j, 0]
        z_ref[...] = z
        tabc_ref[...] = dinv * z

    wspec = lambda shp: pl.BlockSpec(shp, lambda i: tuple(0 for _ in shp))
    return pl.pallas_call(
        body,
        grid=(_TGRID,),
        in_specs=[pl.BlockSpec((2, 3, 8, NCOL), lambda i: (0, 0, i, 0)),
                  _pln(), _pln3(),
                  wspec((3, 16)), wspec((1, 16)), wspec((16, 1))],
        out_specs=[_pln(), _pln()],
        out_shape=[jax.ShapeDtypeStruct((NROW, NCOL), jnp.float32)] * 2,
    )(vp3, dinv, x3, W1, b1r, W2)


def _t3(wp, dinv, z, b2r):
    def body(wp_ref, dinv_ref, z_ref, b2_ref, out_ref):
        dinv = dinv_ref[...]
        out_ref[...] = (dinv * (wp_ref[0] + wp_ref[1])
                        + dinv * dinv * z_ref[...] + b2_ref[0, 0])

    return pl.pallas_call(
        body,
        grid=(_TGRID,),
        in_specs=[_pln2(), _pln(), _pln(),
                  pl.BlockSpec((1, 1), lambda i: (0, 0))],
        out_specs=_pln(),
        out_shape=jax.ShapeDtypeStruct((NROW, NCOL), jnp.float32),
    )(wp, dinv, z, b2r)


def kernel(x, edge_index, W1, b1, W2, b2):
    ei = edge_index.astype(jnp.int32)
    epad = jnp.full((EPAD - N_EDGES,), N_NODES, jnp.int32)
    src = jnp.concatenate([ei[0], epad]).reshape(EROWS, BLK)
    dst = jnp.concatenate([ei[1], epad]).reshape(EROWS, BLK)
    def _r(a):
        t = jax.lax.bitcast_convert_type(a, jnp.uint32)
        t = ((t + jnp.uint32(0x7FFF) + ((t >> 16) & jnp.uint32(1)))
             & jnp.uint32(0xFFFF0000))
        return jax.lax.bitcast_convert_type(t, jnp.float32)
    W1 = _r(W1)
    W2 = _r(W2)
    x3 = jnp.pad(x.T, ((0, 0), (0, NPAD - N_NODES)))

    degp = _edge_pass(dst, dst, [])[0].reshape(2, NROW, NCOL)
    dinv, u3 = _t1(degp, x3.reshape(3, NROW, NCOL))
    u_planes = [u3.reshape(3, NPAD)[kk] for kk in range(3)]
    vp = _edge_pass(src, dst, u_planes)
    vp3 = jnp.stack([v.reshape(2, NROW, NCOL) for v in vp], axis=1)
    tabc, z = _t2(vp3, dinv, x3.reshape(3, NROW, NCOL),
                  W1, b1.reshape(1, 16), W2)
    wp = _edge_pass(src, dst, [tabc.reshape(NPAD)])[0].reshape(2, NROW, NCOL)
    out = _t3(wp, dinv, z, b2.reshape(1, 1))
    return out.reshape(NPAD)[:N_NODES].reshape(N_NODES, 1)
